# Initial kernel scaffold; baseline (speedup 1.0000x reference)
import jax
import jax.numpy as jnp
from jax.experimental import pallas as pl


def kernel(x, W, pe):
    raise NotImplementedError("write your pallas kernel here")


# speedup vs baseline: 3.9487x; 3.9487x over previous
import functools
import math

import jax
import jax.numpy as jnp
from jax import lax
from jax.experimental import pallas as pl
from jax.experimental.pallas import tpu as pltpu
from jax.experimental.pallas import tpu_sc as plsc

B = 1024
L = 200
D = 128
SCALE = math.sqrt(float(D))

NC = 2
NS = 16
NW = NC * NS
SEQ_PER_W = B // NW
HALF = L // 2
LANES = 16
VECS_PER_ROW = D // LANES

_mesh = plsc.VectorSubcoreMesh(core_axis_name="c", subcore_axis_name="s")


@functools.partial(
    pl.kernel,
    out_type=jax.ShapeDtypeStruct((B * L, D), jnp.float32),
    mesh=_mesh,
    scratch_types=[
        pltpu.VMEM((2, HALF), jnp.int32),
        pltpu.VMEM((L, D), jnp.float32),
        pltpu.VMEM((L, D), jnp.float32),
        pltpu.SemaphoreType.DMA,
    ],
)
def _emb_kernel(x_hbm, w_hbm, pe_hbm, out_hbm, idx_v, rows_v, pe_v, sem):
    wid = lax.axis_index("s") * NC + lax.axis_index("c")

    pltpu.sync_copy(pe_hbm.at[pl.ds(0, L)], pe_v)

    def seq_body(s, carry):
        b = wid * SEQ_PER_W + s
        pltpu.sync_copy(x_hbm.at[pl.ds(b * 2, 2)], idx_v)
        cp0 = pltpu.async_copy(w_hbm.at[idx_v.at[0]], rows_v.at[pl.ds(0, HALF)], sem)
        cp1 = pltpu.async_copy(w_hbm.at[idx_v.at[1]], rows_v.at[pl.ds(HALF, HALF)], sem)
        cp0.wait()
        cp1.wait()

        def row_body(r, carry2):
            for c in range(VECS_PER_ROW):
                sl = pl.ds(c * LANES, LANES)
                rows_v[r, sl] = rows_v[r, sl] * SCALE + pe_v[r, sl]
            return carry2

        lax.fori_loop(0, L, row_body, 0)

        pltpu.sync_copy(rows_v, out_hbm.at[pl.ds(b * L, L)])
        return carry

    lax.fori_loop(0, SEQ_PER_W, seq_body, 0)


def kernel(x, W, pe):
    x2 = x.reshape(B * L // HALF, HALF)
    out = _emb_kernel(x2, W, pe)
    return out.reshape(B, L, D)
